# Initial kernel scaffold; baseline (speedup 1.0000x reference)
import jax
import jax.numpy as jnp
from jax.experimental import pallas as pl


def kernel(constraint_features, edge_indices, edge_features, variable_features, n_cons_per_sample, n_vars_per_sample, c_g1, c_b1, c_W1, c_bb1, c_W2, c_bb2, e_g1, e_b1, e_W1, e_bb1, v_g1, v_b1, v_W1, v_bb1, v_W2, v_bb2, vc_Wm, vc_bm, vc_g, vc_b, vc_Wo, vc_bo, cv_Wm, cv_bm, cv_g, cv_b, cv_Wo, cv_bo):
    raise NotImplementedError("write your pallas kernel here")


# speedup vs baseline: 4.9148x; 4.9148x over previous
import functools

import jax
import jax.numpy as jnp
from jax import lax
from jax.experimental import pallas as pl
from jax.experimental.pallas import tpu as pltpu
from jax.experimental.pallas import tpu_sc as plsc

EPSL = 1e-5
N = 10000
NEDGE = 320000
EMB = 128
K = 125
ROWS = NEDGE // K
NW = 32
RPW = ROWS // NW
RSB = 624
RREM = N - 16 * RSB


def _lrelu(x):
    return jnp.maximum(x, 0.01 * x)


def _ln(x, g, b):
    mu = jnp.mean(x, axis=-1, keepdims=True)
    va = jnp.mean((x - mu) ** 2, axis=-1, keepdims=True)
    return (x - mu) * lax.rsqrt(va + EPSL) * g + b


def _dot(x, w):
    return jnp.dot(x, w, preferred_element_type=jnp.float32)


_BLK = 1000
_GRID = N // _BLK


def _row_spec(d):
    return pl.BlockSpec((_BLK, d), lambda i: (i, 0))


def _full_spec(shape):
    nd = len(shape)
    return pl.BlockSpec(shape, lambda i: (0,) * nd)


def _embed_body(cf, vf, cg, cb, cW1, cb1, cW2, cb2, vg, vb, vW1, vb1, vW2,
                vb2, eb, eW, ebb, Wm, bm, c0_o, v0_o, a1_o, b1_o):
    c = _ln(cf[...], cg[...], cb[...])
    c = _lrelu(_dot(c, cW1[...]) + cb1[...])
    c = _lrelu(_dot(c, cW2[...]) + cb2[...])
    v = _ln(vf[...], vg[...], vb[...])
    v = _lrelu(_dot(v, vW1[...]) + vb1[...])
    v = _lrelu(_dot(v, vW2[...]) + vb2[...])
    econ = _lrelu(eb[0, 0] * eW[...] + ebb[...])
    w = Wm[...]
    bias = _dot(econ, w[EMB:2 * EMB]) + bm[...]
    c0_o[...] = c
    v0_o[...] = v
    a1_o[...] = _dot(c, w[0:EMB]) + bias
    b1_o[...] = _dot(v, w[2 * EMB:3 * EMB])


def _embed(cf, vf, cg, cb, cW1, cb1, cW2, cb2, vg, vb, vW1, vb1, vW2, vb2,
           eb, eW, ebb, Wm, bm):
    nfc = cf.shape[1]
    nfv = vf.shape[1]
    o = jax.ShapeDtypeStruct((N, EMB), jnp.float32)
    return pl.pallas_call(
        _embed_body,
        grid=(_GRID,),
        in_specs=[
            _row_spec(nfc), _row_spec(nfv),
            _full_spec((1, nfc)), _full_spec((1, nfc)),
            _full_spec((nfc, EMB)), _full_spec((1, EMB)),
            _full_spec((EMB, EMB)), _full_spec((1, EMB)),
            _full_spec((1, nfv)), _full_spec((1, nfv)),
            _full_spec((nfv, EMB)), _full_spec((1, EMB)),
            _full_spec((EMB, EMB)), _full_spec((1, EMB)),
            _full_spec((1, 1)), _full_spec((1, EMB)), _full_spec((1, EMB)),
            _full_spec((3 * EMB, EMB)), _full_spec((1, EMB)),
        ],
        out_specs=[_row_spec(EMB)] * 4,
        out_shape=[o, o, o, o],
    )(cf, vf, cg, cb, cW1, cb1, cW2, cb2, vg, vb, vW1, vb1, vW2, vb2,
      eb, eW, ebb, Wm, bm)


def _finish_body(s0, s1, cnt0, cnt1, tgt, other, g, b, Wo, bo, eb, eW, ebb,
                 Wm, bm, newt_o, an_o, bn_o):
    cnt = cnt0[:, 0:1] + cnt1[:, 0:1]
    agg = (s0[...] + s1[...]) / jnp.maximum(cnt, 1.0)
    agg = _ln(agg, g[...], b[...])
    wo = Wo[...]
    newt = _lrelu(_dot(agg, wo[0:EMB]) + _dot(tgt[...], wo[EMB:2 * EMB])
                  + bo[...])
    econ = _lrelu(eb[0, 0] * eW[...] + ebb[...])
    w = Wm[...]
    bias = _dot(econ, w[EMB:2 * EMB]) + bm[...]
    newt_o[...] = newt
    an_o[...] = _dot(other[...], w[0:EMB]) + bias
    bn_o[...] = _dot(newt, w[2 * EMB:3 * EMB])


def _finish_project(s0, s1, cnt0, cnt1, tgt, other, g, b, Wo, bo, eb, eW,
                    ebb, Wm, bm):
    o = jax.ShapeDtypeStruct((N, EMB), jnp.float32)
    return pl.pallas_call(
        _finish_body,
        grid=(_GRID,),
        in_specs=[
            _row_spec(EMB), _row_spec(EMB), _row_spec(16), _row_spec(16),
            _row_spec(EMB), _row_spec(EMB),
            _full_spec((1, EMB)), _full_spec((1, EMB)),
            _full_spec((2 * EMB, EMB)), _full_spec((1, EMB)),
            _full_spec((1, 1)), _full_spec((1, EMB)), _full_spec((1, EMB)),
            _full_spec((3 * EMB, EMB)), _full_spec((1, EMB)),
        ],
        out_specs=[_row_spec(EMB)] * 3,
        out_shape=[o, o, o],
    )(s0, s1, cnt0, cnt1, tgt, other, g, b, Wo, bo, eb, eW, ebb, Wm, bm)


def _final_body(s0, s1, cnt0, cnt1, tgt, res, g, b, Wo, bo, out_o):
    cnt = cnt0[:, 0:1] + cnt1[:, 0:1]
    agg = (s0[...] + s1[...]) / jnp.maximum(cnt, 1.0)
    agg = _ln(agg, g[...], b[...])
    wo = Wo[...]
    out_o[...] = _lrelu(_dot(agg, wo[0:EMB]) + _dot(tgt[...], wo[EMB:2 * EMB])
                        + bo[...]) + res[...]


def _final(s0, s1, cnt0, cnt1, tgt, res, g, b, Wo, bo):
    return pl.pallas_call(
        _final_body,
        grid=(_GRID,),
        in_specs=[
            _row_spec(EMB), _row_spec(EMB), _row_spec(16), _row_spec(16),
            _row_spec(EMB), _row_spec(EMB),
            _full_spec((1, EMB)), _full_spec((1, EMB)),
            _full_spec((2 * EMB, EMB)), _full_spec((1, EMB)),
        ],
        out_specs=_row_spec(EMB),
        out_shape=jax.ShapeDtypeStruct((N, EMB), jnp.float32),
    )(s0, s1, cnt0, cnt1, tgt, res, g, b, Wo, bo)


_MESH = plsc.VectorSubcoreMesh(core_axis_name="c", subcore_axis_name="s")


def _edge_body(a_hbm, b_hbm, it_hbm, is_hbm, s_out, idxt, idxs, arows, brows,
               acc, sema, semb):
    cid = lax.axis_index("c")
    sid = lax.axis_index("s")
    wid = sid * 2 + cid

    def _zrow(r, carry):
        for cc in range(8):
            arows[r, cc * 16:(cc + 1) * 16] = jnp.zeros((16,), jnp.float32)
        return carry
    lax.fori_loop(0, 104, _zrow, 0)
    for t in range(RSB // 104):
        pltpu.sync_copy(arows.at[pl.ds(0, 104)],
                        acc.at[pl.ds(sid * RSB + t * 104, 104)])

    @pl.when(sid == 0)
    def _():
        pltpu.sync_copy(arows.at[pl.ds(0, RREM)],
                        acc.at[pl.ds(16 * RSB, RREM)])
    plsc.subcore_barrier()

    base = wid * RPW

    def _blk(blk, carry):
        pltpu.sync_copy(it_hbm.at[pl.ds(base + blk * 8, 8)], idxt)
        pltpu.sync_copy(is_hbm.at[pl.ds(base + blk * 8, 8)], idxs)

        def _chunk(j, c2):
            cpa = pltpu.async_copy(a_hbm.at[idxt.at[j]], arows, sema)
            cpb = pltpu.async_copy(b_hbm.at[idxs.at[j]], brows, semb)
            cpa.wait()
            cpb.wait()

            def _row(r, c3):
                for cc in range(8):
                    lo = cc * 16
                    x = arows[r, lo:lo + 16] + brows[r, lo:lo + 16]
                    brows[r, lo:lo + 16] = jnp.maximum(x, 0.01 * x)
                return c3
            lax.fori_loop(0, K, _row, 0)
            pltpu.sync_copy(brows, acc.at[idxt.at[j]], add=True)
            return c2
        lax.fori_loop(0, RPW // 10, _chunk, 0)
        return carry
    lax.fori_loop(0, 10, _blk, 0)
    plsc.subcore_barrier()
    pltpu.sync_copy(acc.at[pl.ds(sid * RSB, RSB)],
                    s_out.at[cid, pl.ds(sid * RSB, RSB)])

    @pl.when(sid == 0)
    def _():
        pltpu.sync_copy(acc.at[pl.ds(16 * RSB, RREM)],
                        s_out.at[cid, pl.ds(16 * RSB, RREM)])


@functools.partial(
    pl.kernel,
    out_type=jax.ShapeDtypeStruct((2, N, EMB), jnp.float32),
    mesh=_MESH,
    scratch_types=[
        pltpu.VMEM((8, K), jnp.int32),
        pltpu.VMEM((8, K), jnp.int32),
        pltpu.VMEM((K, EMB), jnp.float32),
        pltpu.VMEM((K, EMB), jnp.float32),
        pltpu.VMEM_SHARED((N, EMB), jnp.float32),
        pltpu.SemaphoreType.DMA,
        pltpu.SemaphoreType.DMA,
    ],
)
def _edge_kernel(a_hbm, b_hbm, it_hbm, is_hbm, s_out, idxt, idxs, arows,
                 brows, acc, sema, semb):
    _edge_body(a_hbm, b_hbm, it_hbm, is_hbm, s_out, idxt, idxs, arows, brows,
               acc, sema, semb)


def _cnt_body(eic_hbm, eiv_hbm, outc, outv, idxt, ones, cacc, vacc):
    cid = lax.axis_index("c")
    sid = lax.axis_index("s")
    wid = sid * 2 + cid

    def _zrow(r, carry):
        ones[r, 0:16] = jnp.zeros((16,), jnp.float32)
        return carry
    lax.fori_loop(0, K, _zrow, 0)
    for t in range(RSB // 104):
        pltpu.sync_copy(ones.at[pl.ds(0, 104)],
                        cacc.at[pl.ds(sid * RSB + t * 104, 104)])
        pltpu.sync_copy(ones.at[pl.ds(0, 104)],
                        vacc.at[pl.ds(sid * RSB + t * 104, 104)])

    @pl.when(sid == 0)
    def _():
        pltpu.sync_copy(ones.at[pl.ds(0, RREM)],
                        cacc.at[pl.ds(16 * RSB, RREM)])
        pltpu.sync_copy(ones.at[pl.ds(0, RREM)],
                        vacc.at[pl.ds(16 * RSB, RREM)])

    def _orow(r, carry):
        ones[r, 0:16] = jnp.ones((16,), jnp.float32)
        return carry
    lax.fori_loop(0, K, _orow, 0)
    plsc.subcore_barrier()

    base = wid * RPW
    for ehbm, acc in ((eic_hbm, cacc), (eiv_hbm, vacc)):
        def _blk(blk, carry, ehbm=ehbm, acc=acc):
            pltpu.sync_copy(ehbm.at[pl.ds(base + blk * 8, 8)], idxt)

            def _chunk(j, c2):
                pltpu.sync_copy(ones, acc.at[idxt.at[j]], add=True)
                return c2
            lax.fori_loop(0, 8, _chunk, 0)
            return carry
        lax.fori_loop(0, RPW // 8, _blk, 0)
    plsc.subcore_barrier()
    pltpu.sync_copy(cacc.at[pl.ds(sid * RSB, RSB)],
                    outc.at[cid, pl.ds(sid * RSB, RSB)])
    pltpu.sync_copy(vacc.at[pl.ds(sid * RSB, RSB)],
                    outv.at[cid, pl.ds(sid * RSB, RSB)])

    @pl.when(sid == 0)
    def _():
        pltpu.sync_copy(cacc.at[pl.ds(16 * RSB, RREM)],
                        outc.at[cid, pl.ds(16 * RSB, RREM)])
        pltpu.sync_copy(vacc.at[pl.ds(16 * RSB, RREM)],
                        outv.at[cid, pl.ds(16 * RSB, RREM)])


@functools.partial(
    pl.kernel,
    out_type=[jax.ShapeDtypeStruct((2, N, 16), jnp.float32),
              jax.ShapeDtypeStruct((2, N, 16), jnp.float32)],
    mesh=_MESH,
    scratch_types=[
        pltpu.VMEM((8, K), jnp.int32),
        pltpu.VMEM((K, 16), jnp.float32),
        pltpu.VMEM_SHARED((N, 16), jnp.float32),
        pltpu.VMEM_SHARED((N, 16), jnp.float32),
    ],
)
def _cnt_kernel(eic_hbm, eiv_hbm, outc, outv, idxt, ones, cacc, vacc):
    _cnt_body(eic_hbm, eiv_hbm, outc, outv, idxt, ones, cacc, vacc)


def kernel(constraint_features, edge_indices, edge_features,
           variable_features, n_cons_per_sample, n_vars_per_sample,
           c_g1, c_b1, c_W1, c_bb1, c_W2, c_bb2,
           e_g1, e_b1, e_W1, e_bb1,
           v_g1, v_b1, v_W1, v_bb1, v_W2, v_bb2,
           vc_Wm, vc_bm, vc_g, vc_b, vc_Wo, vc_bo,
           cv_Wm, cv_bm, cv_g, cv_b, cv_Wo, cv_bo):
    r = lambda x: x.reshape(1, -1)
    eic = edge_indices[0].reshape(ROWS, K)
    eiv = edge_indices[1].reshape(ROWS, K)

    c0, v0, a1, b1 = _embed(
        constraint_features, variable_features,
        r(c_g1), r(c_b1), c_W1, r(c_bb1), c_W2, r(c_bb2),
        r(v_g1), r(v_b1), v_W1, r(v_bb1), v_W2, r(v_bb2),
        r(e_b1), e_W1, r(e_bb1), vc_Wm, r(vc_bm))

    cntc, cntv = _cnt_kernel(eic, eiv)

    s = _edge_kernel(a1, b1, eic, eiv)
    c1, a2, b2 = _finish_project(
        s[0], s[1], cntc[0], cntc[1], c0, v0,
        r(vc_g), r(vc_b), vc_Wo, r(vc_bo),
        r(e_b1), e_W1, r(e_bb1), cv_Wm, r(cv_bm))

    s = _edge_kernel(a2, b2, eiv, eic)
    v1, a3, b3 = _finish_project(
        s[0], s[1], cntv[0], cntv[1], v0, c1,
        r(cv_g), r(cv_b), cv_Wo, r(cv_bo),
        r(e_b1), e_W1, r(e_bb1), vc_Wm, r(vc_bm))

    s = _edge_kernel(a3, b3, eic, eiv)
    c2, a4, b4 = _finish_project(
        s[0], s[1], cntc[0], cntc[1], c1, v1,
        r(vc_g), r(vc_b), vc_Wo, r(vc_bo),
        r(e_b1), e_W1, r(e_bb1), cv_Wm, r(cv_bm))

    s = _edge_kernel(a4, b4, eiv, eic)
    return _final(s[0], s[1], cntv[0], cntv[1], v1, v0,
                  r(cv_g), r(cv_b), cv_Wo, r(cv_bo))


# speedup vs baseline: 6.9340x; 1.4108x over previous
import functools

import jax
import jax.numpy as jnp
from jax import lax
from jax.experimental import pallas as pl
from jax.experimental.pallas import tpu as pltpu
from jax.experimental.pallas import tpu_sc as plsc

EPSL = 1e-5
N = 10000
NEDGE = 320000
EMB = 128
H = 64
K = 125
ROWS = NEDGE // K
RPS = ROWS // 16
NT = RPS // 2
RSB = 624
RREM = N - 16 * RSB


def _lrelu(x):
    return jnp.maximum(x, 0.01 * x)


def _ln(x, g, b):
    mu = jnp.mean(x, axis=-1, keepdims=True)
    va = jnp.mean((x - mu) ** 2, axis=-1, keepdims=True)
    return (x - mu) * lax.rsqrt(va + EPSL) * g + b


def _dot(x, w):
    return jnp.dot(x.astype(jnp.bfloat16), w.astype(jnp.bfloat16),
                   preferred_element_type=jnp.float32)


_BLK = 1000
_GRID = N // _BLK


def _row_spec(d):
    return pl.BlockSpec((_BLK, d), lambda i: (i, 0))


def _half_spec():
    return pl.BlockSpec((2, _BLK, H), lambda i: (0, i, 0))


def _full_spec(shape):
    nd = len(shape)
    return pl.BlockSpec(shape, lambda i: (0,) * nd)


def _embed_body(cf, vf, cg, cb, cW1, cb1, cW2, cb2, vg, vb, vW1, vb1, vW2,
                vb2, eb, eW, ebb, Wm, bm, c0_o, v0_o, a1_o, b1_o):
    c = _ln(cf[...], cg[...], cb[...])
    c = _lrelu(_dot(c, cW1[...]) + cb1[...])
    c = _lrelu(_dot(c, cW2[...]) + cb2[...])
    v = _ln(vf[...], vg[...], vb[...])
    v = _lrelu(_dot(v, vW1[...]) + vb1[...])
    v = _lrelu(_dot(v, vW2[...]) + vb2[...])
    econ = _lrelu(eb[0, 0] * eW[...] + ebb[...])
    w = Wm[...]
    bias = _dot(econ, w[EMB:2 * EMB]) + bm[...]
    a = _dot(c, w[0:EMB]) + bias
    b = _dot(v, w[2 * EMB:3 * EMB])
    c0_o[...] = c
    v0_o[...] = v
    a1_o[0] = a[:, 0:H]
    a1_o[1] = a[:, H:EMB]
    b1_o[0] = b[:, 0:H]
    b1_o[1] = b[:, H:EMB]


def _embed(cf, vf, cg, cb, cW1, cb1, cW2, cb2, vg, vb, vW1, vb1, vW2, vb2,
           eb, eW, ebb, Wm, bm):
    nfc = cf.shape[1]
    nfv = vf.shape[1]
    o = jax.ShapeDtypeStruct((N, EMB), jnp.float32)
    oh = jax.ShapeDtypeStruct((2, N, H), jnp.float32)
    return pl.pallas_call(
        _embed_body,
        grid=(_GRID,),
        in_specs=[
            _row_spec(nfc), _row_spec(nfv),
            _full_spec((1, nfc)), _full_spec((1, nfc)),
            _full_spec((nfc, EMB)), _full_spec((1, EMB)),
            _full_spec((EMB, EMB)), _full_spec((1, EMB)),
            _full_spec((1, nfv)), _full_spec((1, nfv)),
            _full_spec((nfv, EMB)), _full_spec((1, EMB)),
            _full_spec((EMB, EMB)), _full_spec((1, EMB)),
            _full_spec((1, 1)), _full_spec((1, EMB)), _full_spec((1, EMB)),
            _full_spec((3 * EMB, EMB)), _full_spec((1, EMB)),
        ],
        out_specs=[_row_spec(EMB), _row_spec(EMB), _half_spec(),
                   _half_spec()],
        out_shape=[o, o, oh, oh],
    )(cf, vf, cg, cb, cW1, cb1, cW2, cb2, vg, vb, vW1, vb1, vW2, vb2,
      eb, eW, ebb, Wm, bm)


def _finish_body(s, cnt, tgt, other, g, b, Wo, bo, eb, eW, ebb,
                 Wm, bm, newt_o, an_o, bn_o):
    agg = jnp.concatenate([s[0], s[1]], axis=-1)
    agg = agg / jnp.maximum(cnt[:, 0:1], 1.0)
    agg = _ln(agg, g[...], b[...])
    wo = Wo[...]
    newt = _lrelu(_dot(agg, wo[0:EMB]) + _dot(tgt[...], wo[EMB:2 * EMB])
                  + bo[...])
    econ = _lrelu(eb[0, 0] * eW[...] + ebb[...])
    w = Wm[...]
    bias = _dot(econ, w[EMB:2 * EMB]) + bm[...]
    a = _dot(other[...], w[0:EMB]) + bias
    bn = _dot(newt, w[2 * EMB:3 * EMB])
    newt_o[...] = newt
    an_o[0] = a[:, 0:H]
    an_o[1] = a[:, H:EMB]
    bn_o[0] = bn[:, 0:H]
    bn_o[1] = bn[:, H:EMB]


def _finish_project(s, cnt, tgt, other, g, b, Wo, bo, eb, eW, ebb, Wm, bm):
    o = jax.ShapeDtypeStruct((N, EMB), jnp.float32)
    oh = jax.ShapeDtypeStruct((2, N, H), jnp.float32)
    return pl.pallas_call(
        _finish_body,
        grid=(_GRID,),
        in_specs=[
            _half_spec(), _row_spec(16),
            _row_spec(EMB), _row_spec(EMB),
            _full_spec((1, EMB)), _full_spec((1, EMB)),
            _full_spec((2 * EMB, EMB)), _full_spec((1, EMB)),
            _full_spec((1, 1)), _full_spec((1, EMB)), _full_spec((1, EMB)),
            _full_spec((3 * EMB, EMB)), _full_spec((1, EMB)),
        ],
        out_specs=[_row_spec(EMB), _half_spec(), _half_spec()],
        out_shape=[o, oh, oh],
    )(s, cnt, tgt, other, g, b, Wo, bo, eb, eW, ebb, Wm, bm)


def _final_body(s, cnt, tgt, res, g, b, Wo, bo, out_o):
    agg = jnp.concatenate([s[0], s[1]], axis=-1)
    agg = agg / jnp.maximum(cnt[:, 0:1], 1.0)
    agg = _ln(agg, g[...], b[...])
    wo = Wo[...]
    out_o[...] = _lrelu(_dot(agg, wo[0:EMB]) + _dot(tgt[...], wo[EMB:2 * EMB])
                        + bo[...]) + res[...]


def _final(s, cnt, tgt, res, g, b, Wo, bo):
    return pl.pallas_call(
        _final_body,
        grid=(_GRID,),
        in_specs=[
            _half_spec(), _row_spec(16),
            _row_spec(EMB), _row_spec(EMB),
            _full_spec((1, EMB)), _full_spec((1, EMB)),
            _full_spec((2 * EMB, EMB)), _full_spec((1, EMB)),
        ],
        out_specs=_row_spec(EMB),
        out_shape=jax.ShapeDtypeStruct((N, EMB), jnp.float32),
    )(s, cnt, tgt, res, g, b, Wo, bo)


_MESH = plsc.VectorSubcoreMesh(core_axis_name="c", subcore_axis_name="s")


def _edge_body(a_hbm, b_hbm, it_hbm, is_hbm, s_out, idxt, idxs,
               a0, b0, m0, a1, b1, m1, acc, sga0, sgb0, sga1, sgb1,
               ssc0, ssc1):
    cid = lax.axis_index("c")
    sid = lax.axis_index("s")

    def _zrow(r, carry):
        for cc in range(H // 16):
            m0[r, cc * 16:(cc + 1) * 16] = jnp.zeros((16,), jnp.float32)
        return carry
    lax.fori_loop(0, 104, _zrow, 0)
    for t in range(RSB // 104):
        pltpu.sync_copy(m0.at[pl.ds(0, 104)],
                        acc.at[pl.ds(sid * RSB + t * 104, 104)])

    @pl.when(sid == 0)
    def _():
        pltpu.sync_copy(m0.at[pl.ds(0, RREM)],
                        acc.at[pl.ds(16 * RSB, RREM)])
    plsc.subcore_barrier()

    base = sid * RPS
    pltpu.sync_copy(it_hbm.at[pl.ds(base, RPS)], idxt)
    pltpu.sync_copy(is_hbm.at[pl.ds(base, RPS)], idxs)
    ah = a_hbm.at[cid]
    bh = b_hbm.at[cid]

    def _gather(j, abuf, bbuf, sa, sb):
        pltpu.async_copy(ah.at[idxt.at[j]], abuf, sa)
        pltpu.async_copy(bh.at[idxs.at[j]], bbuf, sb)

    def _waitg(abuf, bbuf, sa, sb):
        pltpu.make_async_copy(ah.at[idxt.at[0]], abuf, sa).wait()
        pltpu.make_async_copy(bh.at[idxs.at[0]], bbuf, sb).wait()

    def _waitsc(mbuf, ssc):
        pltpu.make_async_copy(mbuf.at[pl.ds(0, K)], acc.at[idxt.at[0]],
                              ssc).wait()

    def _compute(abuf, bbuf, mbuf):
        def _row(r, c3):
            for cc in range(H // 16):
                lo = cc * 16
                x = abuf[r, lo:lo + 16] + bbuf[r, lo:lo + 16]
                mbuf[r, lo:lo + 16] = jnp.maximum(x, 0.01 * x)
            return c3
        lax.fori_loop(0, K, _row, 0)

    _gather(0, a0, b0, sga0, sgb0)
    _gather(1, a1, b1, sga1, sgb1)

    def _iter(t, carry):
        j0 = 2 * t
        _waitg(a0, b0, sga0, sgb0)

        @pl.when(t > 0)
        def _():
            _waitsc(m0, ssc0)
        _compute(a0, b0, m0)
        pltpu.async_copy(m0.at[pl.ds(0, K)], acc.at[idxt.at[j0]], ssc0,
                         add=True)

        @pl.when(t < NT - 1)
        def _():
            _gather(j0 + 2, a0, b0, sga0, sgb0)
        _waitg(a1, b1, sga1, sgb1)

        @pl.when(t > 0)
        def _():
            _waitsc(m1, ssc1)
        _compute(a1, b1, m1)
        pltpu.async_copy(m1.at[pl.ds(0, K)], acc.at[idxt.at[j0 + 1]], ssc1,
                         add=True)

        @pl.when(t < NT - 1)
        def _():
            _gather(j0 + 3, a1, b1, sga1, sgb1)
        return carry
    lax.fori_loop(0, NT, _iter, 0)
    _waitsc(m0, ssc0)
    _waitsc(m1, ssc1)
    plsc.subcore_barrier()
    pltpu.sync_copy(acc.at[pl.ds(sid * RSB, RSB)],
                    s_out.at[cid, pl.ds(sid * RSB, RSB)])

    @pl.when(sid == 0)
    def _():
        pltpu.sync_copy(acc.at[pl.ds(16 * RSB, RREM)],
                        s_out.at[cid, pl.ds(16 * RSB, RREM)])


@functools.partial(
    pl.kernel,
    out_type=jax.ShapeDtypeStruct((2, N, H), jnp.float32),
    mesh=_MESH,
    compiler_params=pltpu.CompilerParams(use_tc_tiling_on_sc=False),
    scratch_types=[
        pltpu.VMEM((RPS, K), jnp.int32),
        pltpu.VMEM((RPS, K), jnp.int32),
        pltpu.VMEM((K, H), jnp.float32),
        pltpu.VMEM((K, H), jnp.float32),
        pltpu.VMEM((K, H), jnp.float32),
        pltpu.VMEM((K, H), jnp.float32),
        pltpu.VMEM((K, H), jnp.float32),
        pltpu.VMEM((K, H), jnp.float32),
        pltpu.VMEM_SHARED((N, H), jnp.float32),
        pltpu.SemaphoreType.DMA,
        pltpu.SemaphoreType.DMA,
        pltpu.SemaphoreType.DMA,
        pltpu.SemaphoreType.DMA,
        pltpu.SemaphoreType.DMA,
        pltpu.SemaphoreType.DMA,
    ],
)
def _edge_kernel(a_hbm, b_hbm, it_hbm, is_hbm, s_out, idxt, idxs,
                 a0, b0, m0, a1, b1, m1, acc, sga0, sgb0, sga1, sgb1,
                 ssc0, ssc1):
    _edge_body(a_hbm, b_hbm, it_hbm, is_hbm, s_out, idxt, idxs,
               a0, b0, m0, a1, b1, m1, acc, sga0, sgb0, sga1, sgb1,
               ssc0, ssc1)


def _cnt_body(eic_hbm, eiv_hbm, outc, outv, idxt, ones, hacc):
    cid = lax.axis_index("c")
    sid = lax.axis_index("s")

    def _zrow(r, carry):
        ones[r, 0:16] = jnp.zeros((16,), jnp.float32)
        return carry
    lax.fori_loop(0, 104, _zrow, 0)
    for t in range(RSB // 104):
        pltpu.sync_copy(ones.at[pl.ds(0, 104)],
                        hacc.at[pl.ds(sid * RSB + t * 104, 104)])

    @pl.when(sid == 0)
    def _():
        pltpu.sync_copy(ones.at[pl.ds(0, RREM)],
                        hacc.at[pl.ds(16 * RSB, RREM)])

    def _orow(r, carry):
        ones[r, 0:16] = jnp.ones((16,), jnp.float32)
        return carry
    lax.fori_loop(0, K, _orow, 0)
    plsc.subcore_barrier()

    base = sid * RPS

    def _run(ehbm):
        pltpu.sync_copy(ehbm.at[pl.ds(base, RPS)], idxt)

        def _chunk(j, carry):
            pltpu.sync_copy(ones.at[pl.ds(0, K)], hacc.at[idxt.at[j]],
                            add=True)
            return carry
        lax.fori_loop(0, RPS, _chunk, 0)

    @pl.when(cid == 0)
    def _():
        _run(eic_hbm)

    @pl.when(cid == 1)
    def _():
        _run(eiv_hbm)
    plsc.subcore_barrier()

    @pl.when(cid == 0)
    def _():
        pltpu.sync_copy(hacc.at[pl.ds(sid * RSB, RSB)],
                        outc.at[pl.ds(sid * RSB, RSB)])

        @pl.when(sid == 0)
        def _():
            pltpu.sync_copy(hacc.at[pl.ds(16 * RSB, RREM)],
                            outc.at[pl.ds(16 * RSB, RREM)])

    @pl.when(cid == 1)
    def _():
        pltpu.sync_copy(hacc.at[pl.ds(sid * RSB, RSB)],
                        outv.at[pl.ds(sid * RSB, RSB)])

        @pl.when(sid == 0)
        def _():
            pltpu.sync_copy(hacc.at[pl.ds(16 * RSB, RREM)],
                            outv.at[pl.ds(16 * RSB, RREM)])


@functools.partial(
    pl.kernel,
    out_type=[jax.ShapeDtypeStruct((N, 16), jnp.float32),
              jax.ShapeDtypeStruct((N, 16), jnp.float32)],
    mesh=_MESH,
    scratch_types=[
        pltpu.VMEM((RPS, K), jnp.int32),
        pltpu.VMEM((104, 16), jnp.float32),
        pltpu.VMEM_SHARED((N, 16), jnp.float32),
    ],
)
def _cnt_kernel(eic_hbm, eiv_hbm, outc, outv, idxt, ones, hacc):
    _cnt_body(eic_hbm, eiv_hbm, outc, outv, idxt, ones, hacc)


def kernel(constraint_features, edge_indices, edge_features,
           variable_features, n_cons_per_sample, n_vars_per_sample,
           c_g1, c_b1, c_W1, c_bb1, c_W2, c_bb2,
           e_g1, e_b1, e_W1, e_bb1,
           v_g1, v_b1, v_W1, v_bb1, v_W2, v_bb2,
           vc_Wm, vc_bm, vc_g, vc_b, vc_Wo, vc_bo,
           cv_Wm, cv_bm, cv_g, cv_b, cv_Wo, cv_bo):
    r = lambda x: x.reshape(1, -1)
    eic = edge_indices[0].reshape(ROWS, K)
    eiv = edge_indices[1].reshape(ROWS, K)

    c0, v0, a1, b1 = _embed(
        constraint_features, variable_features,
        r(c_g1), r(c_b1), c_W1, r(c_bb1), c_W2, r(c_bb2),
        r(v_g1), r(v_b1), v_W1, r(v_bb1), v_W2, r(v_bb2),
        r(e_b1), e_W1, r(e_bb1), vc_Wm, r(vc_bm))

    cntc, cntv = _cnt_kernel(eic, eiv)

    s = _edge_kernel(a1, b1, eic, eiv)
    c1, a2, b2 = _finish_project(
        s, cntc, c0, v0,
        r(vc_g), r(vc_b), vc_Wo, r(vc_bo),
        r(e_b1), e_W1, r(e_bb1), cv_Wm, r(cv_bm))

    s = _edge_kernel(a2, b2, eiv, eic)
    v1, a3, b3 = _finish_project(
        s, cntv, v0, c1,
        r(cv_g), r(cv_b), cv_Wo, r(cv_bo),
        r(e_b1), e_W1, r(e_bb1), vc_Wm, r(vc_bm))

    s = _edge_kernel(a3, b3, eic, eiv)
    c2, a4, b4 = _finish_project(
        s, cntc, c1, v1,
        r(vc_g), r(vc_b), vc_Wo, r(vc_bo),
        r(e_b1), e_W1, r(e_bb1), cv_Wm, r(cv_bm))

    s = _edge_kernel(a4, b4, eiv, eic)
    return _final(s, cntv, v1, v0,
                  r(cv_g), r(cv_b), cv_Wo, r(cv_bo))


# speedup vs baseline: 6.9980x; 1.0092x over previous
import functools

import jax
import jax.numpy as jnp
from jax import lax
from jax.experimental import pallas as pl
from jax.experimental.pallas import tpu as pltpu
from jax.experimental.pallas import tpu_sc as plsc

EPSL = 1e-5
N = 10000
NEDGE = 320000
EMB = 128
H = 64
K = 125
ROWS = NEDGE // K
RPS = ROWS // 16
NT = RPS // 2
RSB = 624
RREM = N - 16 * RSB


def _lrelu(x):
    return jnp.maximum(x, 0.01 * x)


def _ln(x, g, b):
    mu = jnp.mean(x, axis=-1, keepdims=True)
    va = jnp.mean((x - mu) ** 2, axis=-1, keepdims=True)
    return (x - mu) * lax.rsqrt(va + EPSL) * g + b


def _dot(x, w):
    return jnp.dot(x, w, preferred_element_type=jnp.float32)


_BLK = 1000
_GRID = N // _BLK


def _row_spec(d):
    return pl.BlockSpec((_BLK, d), lambda i: (i, 0))


def _half_spec():
    return pl.BlockSpec((2, _BLK, H), lambda i: (0, i, 0))


def _full_spec(shape):
    nd = len(shape)
    return pl.BlockSpec(shape, lambda i: (0,) * nd)


def _embed_body(cf, vf, cg, cb, cW1, cb1, cW2, cb2, vg, vb, vW1, vb1, vW2,
                vb2, eb, eW, ebb, Wm, bm, c0_o, v0_o, a1_o, b1_o):
    c = _ln(cf[...], cg[...], cb[...])
    c = _lrelu(_dot(c, cW1[...]) + cb1[...])
    c = _lrelu(_dot(c, cW2[...]) + cb2[...])
    v = _ln(vf[...], vg[...], vb[...])
    v = _lrelu(_dot(v, vW1[...]) + vb1[...])
    v = _lrelu(_dot(v, vW2[...]) + vb2[...])
    econ = _lrelu(eb[0, 0] * eW[...] + ebb[...])
    w = Wm[...]
    bias = _dot(econ, w[EMB:2 * EMB]) + bm[...]
    a = _dot(c, w[0:EMB]) + bias
    b = _dot(v, w[2 * EMB:3 * EMB])
    c0_o[...] = c
    v0_o[...] = v
    a1_o[0] = a[:, 0:H]
    a1_o[1] = a[:, H:EMB]
    b1_o[0] = b[:, 0:H]
    b1_o[1] = b[:, H:EMB]


def _embed(cf, vf, cg, cb, cW1, cb1, cW2, cb2, vg, vb, vW1, vb1, vW2, vb2,
           eb, eW, ebb, Wm, bm):
    nfc = cf.shape[1]
    nfv = vf.shape[1]
    o = jax.ShapeDtypeStruct((N, EMB), jnp.float32)
    oh = jax.ShapeDtypeStruct((2, N, H), jnp.float32)
    return pl.pallas_call(
        _embed_body,
        grid=(_GRID,),
        in_specs=[
            _row_spec(nfc), _row_spec(nfv),
            _full_spec((1, nfc)), _full_spec((1, nfc)),
            _full_spec((nfc, EMB)), _full_spec((1, EMB)),
            _full_spec((EMB, EMB)), _full_spec((1, EMB)),
            _full_spec((1, nfv)), _full_spec((1, nfv)),
            _full_spec((nfv, EMB)), _full_spec((1, EMB)),
            _full_spec((EMB, EMB)), _full_spec((1, EMB)),
            _full_spec((1, 1)), _full_spec((1, EMB)), _full_spec((1, EMB)),
            _full_spec((3 * EMB, EMB)), _full_spec((1, EMB)),
        ],
        out_specs=[_row_spec(EMB), _row_spec(EMB), _half_spec(),
                   _half_spec()],
        out_shape=[o, o, oh, oh],
    )(cf, vf, cg, cb, cW1, cb1, cW2, cb2, vg, vb, vW1, vb1, vW2, vb2,
      eb, eW, ebb, Wm, bm)


def _finish_body(s, cnt, tgt, other, g, b, Wo, bo, eb, eW, ebb,
                 Wm, bm, newt_o, an_o, bn_o):
    agg = jnp.concatenate([s[0], s[1]], axis=-1)
    agg = agg / jnp.maximum(cnt[:, 0:1], 1.0)
    agg = _ln(agg, g[...], b[...])
    wo = Wo[...]
    newt = _lrelu(_dot(agg, wo[0:EMB]) + _dot(tgt[...], wo[EMB:2 * EMB])
                  + bo[...])
    econ = _lrelu(eb[0, 0] * eW[...] + ebb[...])
    w = Wm[...]
    bias = _dot(econ, w[EMB:2 * EMB]) + bm[...]
    a = _dot(other[...], w[0:EMB]) + bias
    bn = _dot(newt, w[2 * EMB:3 * EMB])
    newt_o[...] = newt
    an_o[0] = a[:, 0:H]
    an_o[1] = a[:, H:EMB]
    bn_o[0] = bn[:, 0:H]
    bn_o[1] = bn[:, H:EMB]


def _finish_project(s, cnt, tgt, other, g, b, Wo, bo, eb, eW, ebb, Wm, bm):
    o = jax.ShapeDtypeStruct((N, EMB), jnp.float32)
    oh = jax.ShapeDtypeStruct((2, N, H), jnp.float32)
    return pl.pallas_call(
        _finish_body,
        grid=(_GRID,),
        in_specs=[
            _half_spec(), _row_spec(16),
            _row_spec(EMB), _row_spec(EMB),
            _full_spec((1, EMB)), _full_spec((1, EMB)),
            _full_spec((2 * EMB, EMB)), _full_spec((1, EMB)),
            _full_spec((1, 1)), _full_spec((1, EMB)), _full_spec((1, EMB)),
            _full_spec((3 * EMB, EMB)), _full_spec((1, EMB)),
        ],
        out_specs=[_row_spec(EMB), _half_spec(), _half_spec()],
        out_shape=[o, oh, oh],
    )(s, cnt, tgt, other, g, b, Wo, bo, eb, eW, ebb, Wm, bm)


def _final_body(s, cnt, tgt, res, g, b, Wo, bo, out_o):
    agg = jnp.concatenate([s[0], s[1]], axis=-1)
    agg = agg / jnp.maximum(cnt[:, 0:1], 1.0)
    agg = _ln(agg, g[...], b[...])
    wo = Wo[...]
    out_o[...] = _lrelu(_dot(agg, wo[0:EMB]) + _dot(tgt[...], wo[EMB:2 * EMB])
                        + bo[...]) + res[...]


def _final(s, cnt, tgt, res, g, b, Wo, bo):
    return pl.pallas_call(
        _final_body,
        grid=(_GRID,),
        in_specs=[
            _half_spec(), _row_spec(16),
            _row_spec(EMB), _row_spec(EMB),
            _full_spec((1, EMB)), _full_spec((1, EMB)),
            _full_spec((2 * EMB, EMB)), _full_spec((1, EMB)),
        ],
        out_specs=_row_spec(EMB),
        out_shape=jax.ShapeDtypeStruct((N, EMB), jnp.float32),
    )(s, cnt, tgt, res, g, b, Wo, bo)


_MESH = plsc.VectorSubcoreMesh(core_axis_name="c", subcore_axis_name="s")


def _edge_body(a_hbm, b_hbm, it_hbm, is_hbm, s_out, idxt, idxs,
               a0, b0, m0, a1, b1, m1, acc, sga0, sgb0, sga1, sgb1,
               ssc0, ssc1):
    cid = lax.axis_index("c")
    sid = lax.axis_index("s")

    def _zrow(r, carry):
        for cc in range(H // 16):
            m0[r, cc * 16:(cc + 1) * 16] = jnp.zeros((16,), jnp.float32)
        return carry
    lax.fori_loop(0, 104, _zrow, 0)
    for t in range(RSB // 104):
        pltpu.sync_copy(m0.at[pl.ds(0, 104)],
                        acc.at[pl.ds(sid * RSB + t * 104, 104)])

    @pl.when(sid == 0)
    def _():
        pltpu.sync_copy(m0.at[pl.ds(0, RREM)],
                        acc.at[pl.ds(16 * RSB, RREM)])
    plsc.subcore_barrier()

    base = sid * RPS
    pltpu.sync_copy(it_hbm.at[pl.ds(base, RPS)], idxt)
    pltpu.sync_copy(is_hbm.at[pl.ds(base, RPS)], idxs)
    ah = a_hbm.at[cid]
    bh = b_hbm.at[cid]

    def _gather(j, abuf, bbuf, sa, sb):
        pltpu.async_copy(ah.at[idxt.at[j]], abuf, sa)
        pltpu.async_copy(bh.at[idxs.at[j]], bbuf, sb)

    def _waitg(abuf, bbuf, sa, sb):
        pltpu.make_async_copy(ah.at[idxt.at[0]], abuf, sa).wait()
        pltpu.make_async_copy(bh.at[idxs.at[0]], bbuf, sb).wait()

    def _waitsc(mbuf, ssc):
        pltpu.make_async_copy(mbuf.at[pl.ds(0, K)], acc.at[idxt.at[0]],
                              ssc).wait()

    def _compute(abuf, bbuf, mbuf):
        def _row(r, c3):
            for cc in range(H // 16):
                lo = cc * 16
                x = abuf[r, lo:lo + 16] + bbuf[r, lo:lo + 16]
                mbuf[r, lo:lo + 16] = jnp.maximum(x, 0.01 * x)
            return c3
        lax.fori_loop(0, K, _row, 0)

    _gather(0, a0, b0, sga0, sgb0)
    _gather(1, a1, b1, sga1, sgb1)

    def _iter(t, carry):
        j0 = 2 * t
        _waitg(a0, b0, sga0, sgb0)

        @pl.when(t > 0)
        def _():
            _waitsc(m0, ssc0)
        _compute(a0, b0, m0)
        pltpu.async_copy(m0.at[pl.ds(0, K)], acc.at[idxt.at[j0]], ssc0,
                         add=True)

        @pl.when(t < NT - 1)
        def _():
            _gather(j0 + 2, a0, b0, sga0, sgb0)
        _waitg(a1, b1, sga1, sgb1)

        @pl.when(t > 0)
        def _():
            _waitsc(m1, ssc1)
        _compute(a1, b1, m1)
        pltpu.async_copy(m1.at[pl.ds(0, K)], acc.at[idxt.at[j0 + 1]], ssc1,
                         add=True)

        @pl.when(t < NT - 1)
        def _():
            _gather(j0 + 3, a1, b1, sga1, sgb1)
        return carry
    lax.fori_loop(0, NT, _iter, 0)
    _waitsc(m0, ssc0)
    _waitsc(m1, ssc1)
    plsc.subcore_barrier()
    pltpu.sync_copy(acc.at[pl.ds(sid * RSB, RSB)],
                    s_out.at[cid, pl.ds(sid * RSB, RSB)])

    @pl.when(sid == 0)
    def _():
        pltpu.sync_copy(acc.at[pl.ds(16 * RSB, RREM)],
                        s_out.at[cid, pl.ds(16 * RSB, RREM)])


@functools.partial(
    pl.kernel,
    out_type=jax.ShapeDtypeStruct((2, N, H), jnp.float32),
    mesh=_MESH,
    compiler_params=pltpu.CompilerParams(use_tc_tiling_on_sc=False),
    scratch_types=[
        pltpu.VMEM((RPS, K), jnp.int32),
        pltpu.VMEM((RPS, K), jnp.int32),
        pltpu.VMEM((K, H), jnp.float32),
        pltpu.VMEM((K, H), jnp.float32),
        pltpu.VMEM((K, H), jnp.float32),
        pltpu.VMEM((K, H), jnp.float32),
        pltpu.VMEM((K, H), jnp.float32),
        pltpu.VMEM((K, H), jnp.float32),
        pltpu.VMEM_SHARED((N, H), jnp.float32),
        pltpu.SemaphoreType.DMA,
        pltpu.SemaphoreType.DMA,
        pltpu.SemaphoreType.DMA,
        pltpu.SemaphoreType.DMA,
        pltpu.SemaphoreType.DMA,
        pltpu.SemaphoreType.DMA,
    ],
)
def _edge_kernel(a_hbm, b_hbm, it_hbm, is_hbm, s_out, idxt, idxs,
                 a0, b0, m0, a1, b1, m1, acc, sga0, sgb0, sga1, sgb1,
                 ssc0, ssc1):
    _edge_body(a_hbm, b_hbm, it_hbm, is_hbm, s_out, idxt, idxs,
               a0, b0, m0, a1, b1, m1, acc, sga0, sgb0, sga1, sgb1,
               ssc0, ssc1)


def _cnt_body(eic_hbm, eiv_hbm, outc, outv, idxt, ones, hacc):
    cid = lax.axis_index("c")
    sid = lax.axis_index("s")

    def _zrow(r, carry):
        ones[r, 0:16] = jnp.zeros((16,), jnp.float32)
        return carry
    lax.fori_loop(0, 104, _zrow, 0)
    for t in range(RSB // 104):
        pltpu.sync_copy(ones.at[pl.ds(0, 104)],
                        hacc.at[pl.ds(sid * RSB + t * 104, 104)])

    @pl.when(sid == 0)
    def _():
        pltpu.sync_copy(ones.at[pl.ds(0, RREM)],
                        hacc.at[pl.ds(16 * RSB, RREM)])

    def _orow(r, carry):
        ones[r, 0:16] = jnp.ones((16,), jnp.float32)
        return carry
    lax.fori_loop(0, K, _orow, 0)
    plsc.subcore_barrier()

    base = sid * RPS

    def _run(ehbm):
        pltpu.sync_copy(ehbm.at[pl.ds(base, RPS)], idxt)

        def _chunk(j, carry):
            pltpu.sync_copy(ones.at[pl.ds(0, K)], hacc.at[idxt.at[j]],
                            add=True)
            return carry
        lax.fori_loop(0, RPS, _chunk, 0)

    @pl.when(cid == 0)
    def _():
        _run(eic_hbm)

    @pl.when(cid == 1)
    def _():
        _run(eiv_hbm)
    plsc.subcore_barrier()

    @pl.when(cid == 0)
    def _():
        pltpu.sync_copy(hacc.at[pl.ds(sid * RSB, RSB)],
                        outc.at[pl.ds(sid * RSB, RSB)])

        @pl.when(sid == 0)
        def _():
            pltpu.sync_copy(hacc.at[pl.ds(16 * RSB, RREM)],
                            outc.at[pl.ds(16 * RSB, RREM)])

    @pl.when(cid == 1)
    def _():
        pltpu.sync_copy(hacc.at[pl.ds(sid * RSB, RSB)],
                        outv.at[pl.ds(sid * RSB, RSB)])

        @pl.when(sid == 0)
        def _():
            pltpu.sync_copy(hacc.at[pl.ds(16 * RSB, RREM)],
                            outv.at[pl.ds(16 * RSB, RREM)])


@functools.partial(
    pl.kernel,
    out_type=[jax.ShapeDtypeStruct((N, 16), jnp.float32),
              jax.ShapeDtypeStruct((N, 16), jnp.float32)],
    mesh=_MESH,
    scratch_types=[
        pltpu.VMEM((RPS, K), jnp.int32),
        pltpu.VMEM((104, 16), jnp.float32),
        pltpu.VMEM_SHARED((N, 16), jnp.float32),
    ],
)
def _cnt_kernel(eic_hbm, eiv_hbm, outc, outv, idxt, ones, hacc):
    _cnt_body(eic_hbm, eiv_hbm, outc, outv, idxt, ones, hacc)


def kernel(constraint_features, edge_indices, edge_features,
           variable_features, n_cons_per_sample, n_vars_per_sample,
           c_g1, c_b1, c_W1, c_bb1, c_W2, c_bb2,
           e_g1, e_b1, e_W1, e_bb1,
           v_g1, v_b1, v_W1, v_bb1, v_W2, v_bb2,
           vc_Wm, vc_bm, vc_g, vc_b, vc_Wo, vc_bo,
           cv_Wm, cv_bm, cv_g, cv_b, cv_Wo, cv_bo):
    r = lambda x: x.reshape(1, -1)
    eic = edge_indices[0].reshape(ROWS, K)
    eiv = edge_indices[1].reshape(ROWS, K)

    c0, v0, a1, b1 = _embed(
        constraint_features, variable_features,
        r(c_g1), r(c_b1), c_W1, r(c_bb1), c_W2, r(c_bb2),
        r(v_g1), r(v_b1), v_W1, r(v_bb1), v_W2, r(v_bb2),
        r(e_b1), e_W1, r(e_bb1), vc_Wm, r(vc_bm))

    cntc, cntv = _cnt_kernel(eic, eiv)

    s = _edge_kernel(a1, b1, eic, eiv)
    c1, a2, b2 = _finish_project(
        s, cntc, c0, v0,
        r(vc_g), r(vc_b), vc_Wo, r(vc_bo),
        r(e_b1), e_W1, r(e_bb1), cv_Wm, r(cv_bm))

    s = _edge_kernel(a2, b2, eiv, eic)
    v1, a3, b3 = _finish_project(
        s, cntv, v0, c1,
        r(cv_g), r(cv_b), cv_Wo, r(cv_bo),
        r(e_b1), e_W1, r(e_bb1), vc_Wm, r(vc_bm))

    s = _edge_kernel(a3, b3, eic, eiv)
    c2, a4, b4 = _finish_project(
        s, cntc, c1, v1,
        r(vc_g), r(vc_b), vc_Wo, r(vc_bo),
        r(e_b1), e_W1, r(e_bb1), cv_Wm, r(cv_bm))

    s = _edge_kernel(a4, b4, eiv, eic)
    return _final(s, cntv, v1, v0,
                  r(cv_g), r(cv_b), cv_Wo, r(cv_bo))
